# baseline (speedup 1.0000x reference)
import functools

import jax
import jax.numpy as jnp
import numpy as np
from jax import lax
from jax.experimental import pallas as pl
from jax.experimental.pallas import tpu as pltpu
from jax.experimental.pallas import tpu_sc as plsc

_N = 100000
_E = 1600000
_H = 3
_C = 16
_HC = _H * _C

_ROWS = 2048
_NPAD = ((_N + _ROWS - 1) // _ROWS) * _ROWS
_EROWS = 2000

_NT = 32
_EPT = _E // _NT
_CH = 400
_CHA = 400
_NH = _N // 2
_SPT = 3200
_TAIL = _NH - 15 * _SPT
_EPS = _E // 16


def _dense_body(x_ref, w_ref, b_ref, o_ref, *, act):
    y = jnp.dot(x_ref[...], w_ref[...], preferred_element_type=jnp.float32)
    y = y + b_ref[...]
    if act == "tanh":
        y = jnp.tanh(y)
    o_ref[...] = y


def _dense(x, w, b, act=None, rows=_ROWS):
    n, k = x.shape
    m = w.shape[1]
    return pl.pallas_call(
        functools.partial(_dense_body, act=act),
        grid=(n // rows,),
        in_specs=[
            pl.BlockSpec((rows, k), lambda i: (i, 0)),
            pl.BlockSpec((k, m), lambda i: (0, 0)),
            pl.BlockSpec((1, m), lambda i: (0, 0)),
        ],
        out_specs=pl.BlockSpec((rows, m), lambda i: (i, 0)),
        out_shape=jax.ShapeDtypeStruct((n, m), jnp.float32),
    )(x, w, b)


def _score_body(qm, kt, etf, src, dst, ex_out,
                srcv, dstv, qmb, kb, eb, exb, sem1, sem2, sem3):
    wid = lax.axis_index("c") * 16 + lax.axis_index("s")
    tbase = wid * _EPT

    def chunk_body(ci, carry):
        base = tbase + ci * _CH
        pltpu.sync_copy(src.at[pl.ds(base, _CH)], srcv)
        pltpu.sync_copy(dst.at[pl.ds(base, _CH)], dstv)
        cp1 = pltpu.async_copy(qm.at[dstv], qmb, sem1)
        cp2 = pltpu.async_copy(kt.at[srcv], kb, sem2)
        cp3 = pltpu.async_copy(etf.at[pl.ds(base * _HC, _CH * _HC)], eb, sem3)
        cp1.wait()
        cp2.wait()
        cp3.wait()

        def grp(g, c2):
            rows = g * 16 + lax.iota(jnp.int32, 16)
            rows48 = rows * _HC
            s = [jnp.zeros((16,), jnp.float32) for _ in range(_H)]
            for f in range(_HC):
                colf = jnp.full((16,), f, jnp.int32)
                gq = plsc.load_gather(qmb, [rows, colf])
                gk = plsc.load_gather(kb, [rows, colf])
                ge = plsc.load_gather(eb, [rows48 + f])
                s[f // _C] = s[f // _C] + gq * (gk + ge)
            for h in range(_H):
                mh = plsc.load_gather(
                    qmb, [rows, jnp.full((16,), _HC + h, jnp.int32)])
                exb[pl.ds(h * _CH + g * 16, 16)] = jnp.exp(s[h] * 0.25 - mh)
            return c2

        lax.fori_loop(0, _CH // 16, grp, 0)
        for h in range(_H):
            pltpu.sync_copy(exb.at[pl.ds(h * _CH, _CH)],
                            ex_out.at[pl.ds(h * _E + base, _CH)])
        return carry

    lax.fori_loop(0, _EPT // _CH, chunk_body, 0)


def _score_pass(qm, kt, etf, src, dst):
    mesh = plsc.VectorSubcoreMesh(core_axis_name="c", subcore_axis_name="s")
    return pl.kernel(
        _score_body,
        mesh=mesh,
        compiler_params=pltpu.CompilerParams(needs_layout_passes=False, use_tc_tiling_on_sc=False),
        out_type=jax.ShapeDtypeStruct((_H * _E,), jnp.float32),
        scratch_types=[
            pltpu.VMEM((_CH,), jnp.int32),
            pltpu.VMEM((_CH,), jnp.int32),
            pltpu.VMEM((_CH, 64), jnp.float32),
            pltpu.VMEM((_CH, _HC), jnp.float32),
            pltpu.VMEM((_CH * _HC,), jnp.float32),
            pltpu.VMEM((_H * _CH,), jnp.float32),
            pltpu.SemaphoreType.DMA,
            pltpu.SemaphoreType.DMA,
            pltpu.SemaphoreType.DMA,
        ],
    )(qm, kt, etf, src, dst)


def _agg_body(v0, v1, v2, eh0, eh1, eh2, ex, src, dst, ed_out,
              srcv, dstv, vb, ehb, exhb, ex3b, msgb, zb, acc, sem1, sem2):
    cid = lax.axis_index("c")
    sid = lax.axis_index("s")
    tbase = sid * _EPS
    stripe = sid * _SPT
    lo = cid * _NH

    def zrow(i, c):
        zb[i, :] = jnp.zeros((16,), jnp.float32)
        return c
    lax.fori_loop(0, _SPT // 8, zrow, 0)

    lane = lax.iota(jnp.int32, 16)
    vtabs = [v0, v1, v2]
    ehtabs = [eh0, eh1, eh2]
    for ph in range(4):
        @pl.when(sid < 15)
        def _():
            for j in range(8):
                pltpu.sync_copy(
                    zb, acc.at[pl.ds(stripe + j * (_SPT // 8), _SPT // 8)])

        @pl.when(sid == 15)
        def _():
            for j in range(5):
                pltpu.sync_copy(
                    zb.at[pl.ds(0, _TAIL // 5)],
                    acc.at[pl.ds(15 * _SPT + j * (_TAIL // 5), _TAIL // 5)])
        plsc.subcore_barrier()

        def chunk_body(ci, carry, ph=ph):
            base = tbase + ci * _CHA
            pltpu.sync_copy(dst.at[pl.ds(base, _CHA)], dstv)

            def remap(g, c2):
                d = dstv[pl.ds(g * 16, 16)] - lo
                bad = (d < 0) | (d >= _NH)
                dstv[pl.ds(g * 16, 16)] = jnp.where(bad, _NH + lane, d)
                return c2
            lax.fori_loop(0, _CHA // 16, remap, 0)

            if ph < _H:
                pltpu.sync_copy(src.at[pl.ds(base, _CHA)], srcv)
                cp1 = pltpu.async_copy(vtabs[ph].at[srcv], vb, sem1)
                cp2 = pltpu.async_copy(
                    ehtabs[ph].at[pl.ds(base * _C, _CHA * _C)], ehb, sem2)
                pltpu.sync_copy(ex.at[pl.ds(ph * _E + base, _CHA)], exhb)
                cp1.wait()
                cp2.wait()

                lanec = lax.iota(jnp.int32, 16) * _C

                def egrp(g, c2):
                    erows = g * 16 + lane
                    exv = exhb[pl.ds(g * 16, 16)]
                    fbase = g * (16 * _C) + lanec
                    for f in range(_C):
                        colf = jnp.full((16,), f, jnp.int32)
                        vcol = plsc.load_gather(vb, [erows, colf])
                        ecol = plsc.load_gather(ehb, [fbase + f])
                        plsc.store_scatter(
                            msgb, [erows, colf], (vcol + ecol) * exv)
                    return c2
                lax.fori_loop(0, _CHA // 16, egrp, 0, unroll=2)
            else:
                for h in range(_H):
                    pltpu.sync_copy(ex.at[pl.ds(h * _E + base, _CHA)],
                                    ex3b.at[pl.ds(h * _CHA, _CHA)])
                rowsel = jnp.minimum(lane, 2) * _CHA

                def edge(i, c2):
                    g = plsc.load_gather(ex3b, [rowsel + i])
                    msgb[i, :] = jnp.where(lane < _H, g, 0.0)
                    return c2
                lax.fori_loop(0, _CHA, edge, 0, unroll=4)

            pltpu.sync_copy(msgb, acc.at[dstv], add=True)
            return carry

        lax.fori_loop(0, _EPS // _CHA, chunk_body, 0)
        plsc.subcore_barrier()

        @pl.when(sid < 15)
        def _(ph=ph):
            pltpu.sync_copy(
                acc.at[pl.ds(stripe, _SPT)],
                ed_out.at[ph, pl.ds(lo + stripe, _SPT), :])

        @pl.when(sid == 15)
        def _(ph=ph):
            pltpu.sync_copy(
                acc.at[pl.ds(15 * _SPT, _TAIL)],
                ed_out.at[ph, pl.ds(lo + 15 * _SPT, _TAIL), :])
        plsc.subcore_barrier()


def _agg_pass(v0, v1, v2, eh0, eh1, eh2, ex, src, dst):
    mesh = plsc.VectorSubcoreMesh(core_axis_name="c", subcore_axis_name="s")
    return pl.kernel(
        _agg_body,
        mesh=mesh,
        compiler_params=pltpu.CompilerParams(
            needs_layout_passes=False, use_tc_tiling_on_sc=False),
        out_type=jax.ShapeDtypeStruct((4, _NPAD, _C), jnp.float32),
        scratch_types=[
            pltpu.VMEM((_CHA,), jnp.int32),
            pltpu.VMEM((_CHA,), jnp.int32),
            pltpu.VMEM((_CHA, _C), jnp.float32),
            pltpu.VMEM((_CHA * _C,), jnp.float32),
            pltpu.VMEM((_CHA,), jnp.float32),
            pltpu.VMEM((_H * _CHA,), jnp.float32),
            pltpu.VMEM((_CHA, _C), jnp.float32),
            pltpu.VMEM((_SPT // 8, _C), jnp.float32),
            pltpu.VMEM_SHARED((_NH + 16, _C), jnp.float32),
            pltpu.SemaphoreType.DMA,
            pltpu.SemaphoreType.DMA,
        ],
    )(v0, v1, v2, eh0, eh1, eh2, ex, src, dst)


def _combine_body(ed_ref, skip_ref, o_ref):
    y = ed_ref[...]
    msg = jnp.concatenate([y[0], y[1], y[2]], axis=-1)
    den = jnp.concatenate(
        [jnp.broadcast_to(y[3][:, h:h + 1], y[3].shape[:1] + (_C,))
         for h in range(_H)], axis=-1)
    o_ref[...] = jnp.tanh(msg / (den + 1e-30) + skip_ref[...])


def _combine(ed, skip):
    return pl.pallas_call(
        _combine_body,
        grid=(_NPAD // _ROWS,),
        in_specs=[
            pl.BlockSpec((4, _ROWS, _C), lambda i: (0, i, 0)),
            pl.BlockSpec((_ROWS, _HC), lambda i: (i, 0)),
        ],
        out_specs=pl.BlockSpec((_ROWS, _HC), lambda i: (i, 0)),
        out_shape=jax.ShapeDtypeStruct((_NPAD, _HC), jnp.float32),
    )(ed, skip)


def _conv_layer(h, src, dst, edge_attr, max_attr, p, pre):
    qkvs = jnp.concatenate(
        [p[pre + "Wq"], p[pre + "Wk"], p[pre + "Wv"], p[pre + "Wskip"]],
        axis=1)
    bqkvs = jnp.concatenate(
        [p[pre + "bq"], p[pre + "bk"], p[pre + "bv"], p[pre + "bskip"]]
    )[None, :]
    qkvs_out = _dense(h, qkvs, bqkvs)
    q = qkvs_out[:_N, :_HC]
    k = qkvs_out[:_N, _HC:2 * _HC]
    v = qkvs_out[:_N, 2 * _HC:3 * _HC]
    skip = qkvs_out[:, 3 * _HC:]

    et = _dense(edge_attr, p[pre + "We"], p[pre + "be"][None, :], rows=_EROWS)

    qn = jnp.sqrt(jnp.sum(q.reshape(_N, _H, _C) ** 2, axis=-1))
    kmax = jnp.max(
        jnp.sqrt(jnp.sum(k.reshape(_N, _H, _C) ** 2, axis=-1)), axis=0)
    wef = jnp.sqrt(
        jnp.sum(p[pre + "We"].reshape(-1, _H, _C) ** 2, axis=(0, 2)))
    bef = jnp.sqrt(jnp.sum(p[pre + "be"].reshape(_H, _C) ** 2, axis=-1))
    emax = max_attr * wef + bef
    m = qn * (kmax + emax)[None, :] * 0.25

    qm = jnp.concatenate(
        [q, m, jnp.zeros((_N, 64 - _HC - _H), jnp.float32)], axis=1)
    vh = jnp.transpose(v.reshape(_N, _H, _C), (1, 0, 2))

    eh = jnp.transpose(et.reshape(_E, _H, _C), (1, 0, 2)).reshape(_H, _E * _C)

    ex = _score_pass(qm, k, et.reshape(-1), src, dst)
    ed = _agg_pass(vh[0], vh[1], vh[2], eh[0], eh[1], eh[2], ex, src, dst)
    return _combine(ed, skip)


def kernel(x, edge_index, edge_attr_dict, params):
    src = edge_index[0]
    dst = edge_index[1]
    xpad = jnp.pad(x, ((0, _NPAD - _N), (0, 0)))
    h = _dense(xpad, params["lin1_W"], params["lin1_b"][None, :], act="tanh")
    max_attr = jnp.sqrt(jnp.max(jnp.sum(edge_attr_dict ** 2, axis=-1)))
    for l in range(2):
        h = _conv_layer(h, src, dst, edge_attr_dict, max_attr,
                        params, "conv%d_" % l)
    return h[:_N]


# speedup vs baseline: 1.1415x; 1.1415x over previous
import functools

import jax
import jax.numpy as jnp
import numpy as np
from jax import lax
from jax.experimental import pallas as pl
from jax.experimental.pallas import tpu as pltpu
from jax.experimental.pallas import tpu_sc as plsc

_N = 100000
_E = 1600000
_H = 3
_C = 16
_HC = _H * _C

_ROWS = 2048
_NPAD = ((_N + _ROWS - 1) // _ROWS) * _ROWS
_EROWS = 2000

_NT = 32
_EPT = _E // _NT
_CH = 400
_CHA = 400
_NH = _N // 2
_SPT = 3200
_TAIL = _NH - 15 * _SPT
_EPS = _E // 16


def _dense_body(x_ref, w_ref, b_ref, o_ref, *, act):
    y = jnp.dot(x_ref[...], w_ref[...], preferred_element_type=jnp.float32)
    y = y + b_ref[...]
    if act == "tanh":
        y = jnp.tanh(y)
    o_ref[...] = y


def _dense(x, w, b, act=None, rows=_ROWS):
    n, k = x.shape
    m = w.shape[1]
    return pl.pallas_call(
        functools.partial(_dense_body, act=act),
        grid=(n // rows,),
        in_specs=[
            pl.BlockSpec((rows, k), lambda i: (i, 0)),
            pl.BlockSpec((k, m), lambda i: (0, 0)),
            pl.BlockSpec((1, m), lambda i: (0, 0)),
        ],
        out_specs=pl.BlockSpec((rows, m), lambda i: (i, 0)),
        out_shape=jax.ShapeDtypeStruct((n, m), jnp.float32),
    )(x, w, b)


def _score_body(qm, kt, etf, src, dst, ex_out,
                srcv, dstv, qmb, kb, eb, exb, sem1, sem2, sem3):
    wid = lax.axis_index("c") * 16 + lax.axis_index("s")
    tbase = wid * _EPT

    def chunk_body(ci, carry):
        base = tbase + ci * _CH
        pltpu.sync_copy(src.at[pl.ds(base, _CH)], srcv)
        pltpu.sync_copy(dst.at[pl.ds(base, _CH)], dstv)
        cp1 = pltpu.async_copy(qm.at[dstv], qmb, sem1)
        cp2 = pltpu.async_copy(kt.at[srcv], kb, sem2)
        cp3 = pltpu.async_copy(etf.at[pl.ds(base * _HC, _CH * _HC)], eb, sem3)
        cp1.wait()
        cp2.wait()
        cp3.wait()

        def grp(g, c2):
            rows = g * 16 + lax.iota(jnp.int32, 16)
            rows48 = rows * _HC
            s = [jnp.zeros((16,), jnp.float32) for _ in range(_H)]
            for f in range(_HC):
                colf = jnp.full((16,), f, jnp.int32)
                gq = plsc.load_gather(qmb, [rows, colf])
                gk = plsc.load_gather(kb, [rows, colf])
                ge = plsc.load_gather(eb, [rows48 + f])
                s[f // _C] = s[f // _C] + gq * (gk + ge)
            for h in range(_H):
                mh = plsc.load_gather(
                    qmb, [rows, jnp.full((16,), _HC + h, jnp.int32)])
                exb[pl.ds(h * _CH + g * 16, 16)] = jnp.exp(s[h] * 0.25 - mh)
            return c2

        lax.fori_loop(0, _CH // 16, grp, 0)
        for h in range(_H):
            pltpu.sync_copy(exb.at[pl.ds(h * _CH, _CH)],
                            ex_out.at[pl.ds(h * _E + base, _CH)])
        return carry

    lax.fori_loop(0, _EPT // _CH, chunk_body, 0)


def _score_pass(qm, kt, etf, src, dst):
    mesh = plsc.VectorSubcoreMesh(core_axis_name="c", subcore_axis_name="s")
    return pl.kernel(
        _score_body,
        mesh=mesh,
        compiler_params=pltpu.CompilerParams(needs_layout_passes=False, use_tc_tiling_on_sc=False),
        out_type=jax.ShapeDtypeStruct((_H * _E,), jnp.float32),
        scratch_types=[
            pltpu.VMEM((_CH,), jnp.int32),
            pltpu.VMEM((_CH,), jnp.int32),
            pltpu.VMEM((_CH, 64), jnp.float32),
            pltpu.VMEM((_CH, _HC), jnp.float32),
            pltpu.VMEM((_CH * _HC,), jnp.float32),
            pltpu.VMEM((_H * _CH,), jnp.float32),
            pltpu.SemaphoreType.DMA,
            pltpu.SemaphoreType.DMA,
            pltpu.SemaphoreType.DMA,
        ],
    )(qm, kt, etf, src, dst)


def _agg_body(v0, v1, v2, eh0, eh1, eh2, ex, src, dst, ed_out,
              srcv, dstv, vb, ehb, exhb, ex3b, msgb, zb, acc, sem1, sem2):
    cid = lax.axis_index("c")
    sid = lax.axis_index("s")
    tbase = sid * _EPS
    stripe = sid * _SPT
    lo = cid * _NH

    def zrow(i, c):
        zb[i, :] = jnp.zeros((16,), jnp.float32)
        return c
    lax.fori_loop(0, _SPT // 8, zrow, 0)

    lane = lax.iota(jnp.int32, 16)
    vtabs = [v0, v1, v2]
    ehtabs = [eh0, eh1, eh2]
    for ph in range(4):
        @pl.when(sid < 15)
        def _():
            for j in range(8):
                pltpu.sync_copy(
                    zb, acc.at[pl.ds(stripe + j * (_SPT // 8), _SPT // 8)])

        @pl.when(sid == 15)
        def _():
            for j in range(5):
                pltpu.sync_copy(
                    zb.at[pl.ds(0, _TAIL // 5)],
                    acc.at[pl.ds(15 * _SPT + j * (_TAIL // 5), _TAIL // 5)])
        plsc.subcore_barrier()

        def chunk_body(ci, carry, ph=ph):
            base = tbase + ci * _CHA
            pltpu.sync_copy(dst.at[pl.ds(base, _CHA)], dstv)

            def remap(g, c2):
                d = dstv[pl.ds(g * 16, 16)] - lo
                bad = (d < 0) | (d >= _NH)
                dstv[pl.ds(g * 16, 16)] = jnp.where(bad, _NH + lane, d)
                return c2
            lax.fori_loop(0, _CHA // 16, remap, 0, unroll=5)

            if ph < _H:
                pltpu.sync_copy(src.at[pl.ds(base, _CHA)], srcv)
                cp1 = pltpu.async_copy(vtabs[ph].at[srcv], vb, sem1)
                cp2 = pltpu.async_copy(
                    ehtabs[ph].at[pl.ds(base * _C, _CHA * _C)], ehb, sem2)
                pltpu.sync_copy(ex.at[pl.ds(ph * _E + base, _CHA)], exhb)
                cp1.wait()
                cp2.wait()

                def edge(i, c2):
                    exs = plsc.load_gather(
                        exhb, [jnp.full((16,), 0, jnp.int32) + i])
                    msgb[i, :] = (vb[i, :] + ehb[pl.ds(i * _C, _C)]) * exs
                    return c2
                lax.fori_loop(0, _CHA, edge, 0, unroll=8)
            else:
                for h in range(_H):
                    pltpu.sync_copy(ex.at[pl.ds(h * _E + base, _CHA)],
                                    ex3b.at[pl.ds(h * _CHA, _CHA)])
                rowsel = jnp.minimum(lane, 2) * _CHA

                def edge(i, c2):
                    g = plsc.load_gather(ex3b, [rowsel + i])
                    msgb[i, :] = jnp.where(lane < _H, g, 0.0)
                    return c2
                lax.fori_loop(0, _CHA, edge, 0, unroll=8)

            pltpu.sync_copy(msgb, acc.at[dstv], add=True)
            return carry

        lax.fori_loop(0, _EPS // _CHA, chunk_body, 0)
        plsc.subcore_barrier()

        @pl.when(sid < 15)
        def _(ph=ph):
            pltpu.sync_copy(
                acc.at[pl.ds(stripe, _SPT)],
                ed_out.at[ph, pl.ds(lo + stripe, _SPT), :])

        @pl.when(sid == 15)
        def _(ph=ph):
            pltpu.sync_copy(
                acc.at[pl.ds(15 * _SPT, _TAIL)],
                ed_out.at[ph, pl.ds(lo + 15 * _SPT, _TAIL), :])
        plsc.subcore_barrier()


def _agg_pass(v0, v1, v2, eh0, eh1, eh2, ex, src, dst):
    mesh = plsc.VectorSubcoreMesh(core_axis_name="c", subcore_axis_name="s")
    return pl.kernel(
        _agg_body,
        mesh=mesh,
        compiler_params=pltpu.CompilerParams(
            needs_layout_passes=False, use_tc_tiling_on_sc=False),
        out_type=jax.ShapeDtypeStruct((4, _NPAD, _C), jnp.float32),
        scratch_types=[
            pltpu.VMEM((_CHA,), jnp.int32),
            pltpu.VMEM((_CHA,), jnp.int32),
            pltpu.VMEM((_CHA, _C), jnp.float32),
            pltpu.VMEM((_CHA * _C,), jnp.float32),
            pltpu.VMEM((_CHA,), jnp.float32),
            pltpu.VMEM((_H * _CHA,), jnp.float32),
            pltpu.VMEM((_CHA, _C), jnp.float32),
            pltpu.VMEM((_SPT // 8, _C), jnp.float32),
            pltpu.VMEM_SHARED((_NH + 16, _C), jnp.float32),
            pltpu.SemaphoreType.DMA,
            pltpu.SemaphoreType.DMA,
        ],
    )(v0, v1, v2, eh0, eh1, eh2, ex, src, dst)


def _combine_body(ed_ref, skip_ref, o_ref):
    y = ed_ref[...]
    msg = jnp.concatenate([y[0], y[1], y[2]], axis=-1)
    den = jnp.concatenate(
        [jnp.broadcast_to(y[3][:, h:h + 1], y[3].shape[:1] + (_C,))
         for h in range(_H)], axis=-1)
    o_ref[...] = jnp.tanh(msg / (den + 1e-30) + skip_ref[...])


def _combine(ed, skip):
    return pl.pallas_call(
        _combine_body,
        grid=(_NPAD // _ROWS,),
        in_specs=[
            pl.BlockSpec((4, _ROWS, _C), lambda i: (0, i, 0)),
            pl.BlockSpec((_ROWS, _HC), lambda i: (i, 0)),
        ],
        out_specs=pl.BlockSpec((_ROWS, _HC), lambda i: (i, 0)),
        out_shape=jax.ShapeDtypeStruct((_NPAD, _HC), jnp.float32),
    )(ed, skip)


def _conv_layer(h, src, dst, edge_attr, max_attr, p, pre):
    qkvs = jnp.concatenate(
        [p[pre + "Wq"], p[pre + "Wk"], p[pre + "Wv"], p[pre + "Wskip"]],
        axis=1)
    bqkvs = jnp.concatenate(
        [p[pre + "bq"], p[pre + "bk"], p[pre + "bv"], p[pre + "bskip"]]
    )[None, :]
    qkvs_out = _dense(h, qkvs, bqkvs)
    q = qkvs_out[:_N, :_HC]
    k = qkvs_out[:_N, _HC:2 * _HC]
    v = qkvs_out[:_N, 2 * _HC:3 * _HC]
    skip = qkvs_out[:, 3 * _HC:]

    et = _dense(edge_attr, p[pre + "We"], p[pre + "be"][None, :], rows=_EROWS)

    qn = jnp.sqrt(jnp.sum(q.reshape(_N, _H, _C) ** 2, axis=-1))
    kmax = jnp.max(
        jnp.sqrt(jnp.sum(k.reshape(_N, _H, _C) ** 2, axis=-1)), axis=0)
    wef = jnp.sqrt(
        jnp.sum(p[pre + "We"].reshape(-1, _H, _C) ** 2, axis=(0, 2)))
    bef = jnp.sqrt(jnp.sum(p[pre + "be"].reshape(_H, _C) ** 2, axis=-1))
    emax = max_attr * wef + bef
    m = qn * (kmax + emax)[None, :] * 0.25

    qm = jnp.concatenate(
        [q, m, jnp.zeros((_N, 64 - _HC - _H), jnp.float32)], axis=1)
    vh = jnp.transpose(v.reshape(_N, _H, _C), (1, 0, 2))

    eh = jnp.transpose(et.reshape(_E, _H, _C), (1, 0, 2)).reshape(_H, _E * _C)

    ex = _score_pass(qm, k, et.reshape(-1), src, dst)
    ed = _agg_pass(vh[0], vh[1], vh[2], eh[0], eh[1], eh[2], ex, src, dst)
    return _combine(ed, skip)


def kernel(x, edge_index, edge_attr_dict, params):
    src = edge_index[0]
    dst = edge_index[1]
    xpad = jnp.pad(x, ((0, _NPAD - _N), (0, 0)))
    h = _dense(xpad, params["lin1_W"], params["lin1_b"][None, :], act="tanh")
    max_attr = jnp.sqrt(jnp.max(jnp.sum(edge_attr_dict ** 2, axis=-1)))
    for l in range(2):
        h = _conv_layer(h, src, dst, edge_attr_dict, max_attr,
                        params, "conv%d_" % l)
    return h[:_N]
